# Initial kernel scaffold; baseline (speedup 1.0000x reference)
import jax
import jax.numpy as jnp
from jax.experimental import pallas as pl


def kernel(x, timestamp, te):
    raise NotImplementedError("write your pallas kernel here")


# speedup vs baseline: 2.0909x; 2.0909x over previous
import jax
import jax.numpy as jnp
from jax import lax
from jax.experimental import pallas as pl
from jax.experimental.pallas import tpu as pltpu
from jax.experimental.pallas import tpu_sc as plsc

D_MODEL = 64
MAX_LEN = 446
BATCH = 4096
HIST = 200
ROWS = BATCH * HIST
C = 256
CB = C * D_MODEL
LANES = 16


def _make_sc_call():
    mesh = plsc.VectorSubcoreMesh(core_axis_name="c", subcore_axis_name="s")
    nc, ns = mesh.num_cores, mesh.num_subcores
    nw = nc * ns
    assert ROWS % (nw * C) == 0
    rpw = ROWS // nw
    nch = rpw // C
    assert nch % 2 == 0

    def body(x_hbm, ts_hbm, te_hbm, out_hbm,
             te_v, ts0, ts1, xi0, xi1, xo0, xo1,
             ste, si0, si1, so0, so1):
        wid = lax.axis_index("s") * nc + lax.axis_index("c")
        base = wid * rpw

        pltpu.async_copy(te_hbm, te_v, ste).wait()

        def start_in(c, xi, tsb, sem):
            r0 = base + c * C
            pltpu.async_copy(x_hbm.at[pl.ds(r0 * D_MODEL, CB)], xi, sem)
            pltpu.async_copy(ts_hbm.at[pl.ds(r0, C)], tsb, sem)

        def wait_in(xi, tsb, sem):
            pltpu.make_async_copy(x_hbm.at[pl.ds(0, CB)], xi, sem).wait()
            pltpu.make_async_copy(ts_hbm.at[pl.ds(0, C)], tsb, sem).wait()

        def start_out(c, xo, sem):
            r0 = base + c * C
            pltpu.async_copy(xo, out_hbm.at[pl.ds(r0 * D_MODEL, CB)], sem)

        def wait_out(xo, sem):
            pltpu.make_async_copy(x_hbm.at[pl.ds(0, CB)], xo, sem).wait()

        def compute(tsb, xi, xo):
            @pl.loop(0, C, step=LANES)
            def _rows(j0):
                tbv = tsb[pl.ds(j0, LANES)] * D_MODEL
                for jj in range(LANES):
                    tb = tbv[jj]
                    ro = (j0 + jj) * D_MODEL
                    for k in range(0, D_MODEL, LANES):
                        tev = te_v[pl.ds(tb + k, LANES)]
                        xv = xi[pl.ds(ro + k, LANES)]
                        xo[pl.ds(ro + k, LANES)] = xv + tev

        start_in(0, xi0, ts0, si0)
        start_in(1, xi1, ts1, si1)

        bufs = ((xi0, ts0, xo0, si0, so0), (xi1, ts1, xo1, si1, so1))

        @pl.loop(0, nch, step=2)
        def _chunks(c0):
            for b in range(2):
                xi, tsb, xo, si, so = bufs[b]
                c = c0 + b
                wait_in(xi, tsb, si)

                @pl.when(c >= 2)
                def _():
                    wait_out(xo, so)

                compute(tsb, xi, xo)
                start_out(c, xo, so)

                @pl.when(c + 2 < nch)
                def _():
                    start_in(c + 2, xi, tsb, si)

        wait_out(xo0, so0)
        wait_out(xo1, so1)

    f32, i32 = jnp.float32, jnp.int32
    return pl.kernel(
        body,
        out_type=jax.ShapeDtypeStruct((ROWS * D_MODEL,), f32),
        mesh=mesh,
        scratch_types=[
            pltpu.VMEM((MAX_LEN * D_MODEL,), f32),
            pltpu.VMEM((C,), i32),
            pltpu.VMEM((C,), i32),
            pltpu.VMEM((CB,), f32),
            pltpu.VMEM((CB,), f32),
            pltpu.VMEM((CB,), f32),
            pltpu.VMEM((CB,), f32),
            pltpu.SemaphoreType.DMA,
            pltpu.SemaphoreType.DMA,
            pltpu.SemaphoreType.DMA,
            pltpu.SemaphoreType.DMA,
            pltpu.SemaphoreType.DMA,
        ],
    )


def kernel(x, timestamp, te):
    xf = x.reshape(ROWS * D_MODEL)
    tsf = timestamp.astype(jnp.int32).reshape(ROWS)
    tef = te.reshape(MAX_LEN * D_MODEL)
    out = _make_sc_call()(xf, tsf, tef)
    return out.reshape(BATCH, HIST, D_MODEL)


# speedup vs baseline: 2.6823x; 1.2828x over previous
import jax
import jax.numpy as jnp
from jax import lax
from jax.experimental import pallas as pl
from jax.experimental.pallas import tpu as pltpu
from jax.experimental.pallas import tpu_sc as plsc

D_MODEL = 64
MAX_LEN = 446
BATCH = 4096
HIST = 200
ROWS = BATCH * HIST
C = 256
CB = C * D_MODEL
LANES = 16


def _make_sc_call():
    mesh = plsc.VectorSubcoreMesh(core_axis_name="c", subcore_axis_name="s")
    nc, ns = mesh.num_cores, mesh.num_subcores
    nw = nc * ns
    assert ROWS % (nw * C) == 0
    rpw = ROWS // nw
    nch = rpw // C
    main = nch - (nch % 3)

    def body(x_hbm, ts_hbm, te_hbm, out_hbm,
             te_v, ts0, ts1, ts2, xb0, xb1, xb2,
             ste, si0, si1, si2, so0, so1, so2):
        wid = lax.axis_index("s") * nc + lax.axis_index("c")
        base = wid * rpw

        pltpu.async_copy(te_hbm, te_v, ste).wait()

        bufs = ((xb0, ts0, si0, so0), (xb1, ts1, si1, so1), (xb2, ts2, si2, so2))

        def start_in(c, b):
            xb, tsb, si, _ = bufs[b]
            r0 = base + c * C
            pltpu.async_copy(x_hbm.at[pl.ds(r0 * D_MODEL, CB)], xb, si)
            pltpu.async_copy(ts_hbm.at[pl.ds(r0, C)], tsb, si)

        def wait_in(b):
            xb, tsb, si, _ = bufs[b]
            pltpu.make_async_copy(x_hbm.at[pl.ds(0, CB)], xb, si).wait()
            pltpu.make_async_copy(ts_hbm.at[pl.ds(0, C)], tsb, si).wait()

        def start_out(c, b):
            xb, _, _, so = bufs[b]
            r0 = base + c * C
            pltpu.async_copy(xb, out_hbm.at[pl.ds(r0 * D_MODEL, CB)], so)

        def wait_out(b):
            xb, _, _, so = bufs[b]
            pltpu.make_async_copy(x_hbm.at[pl.ds(0, CB)], xb, so).wait()

        def compute(b):
            xb, tsb, _, _ = bufs[b]

            @pl.loop(0, C, step=LANES)
            def _rows(j0):
                tbv = tsb[pl.ds(j0, LANES)] * D_MODEL
                for jj in range(0, LANES, 4):
                    tev = [te_v[pl.ds(tbv[jj + r] + k, LANES)]
                           for r in range(4)
                           for k in range(0, D_MODEL, LANES)]
                    for r in range(4):
                        ro = (j0 + jj + r) * D_MODEL
                        for ki, k in enumerate(range(0, D_MODEL, LANES)):
                            plsc.addupdate(xb.at[pl.ds(ro + k, LANES)],
                                           tev[r * 4 + ki])

        def step(c, b, first):
            wait_in(b)
            compute(b)
            start_out(c, b)
            nxt = c + 2

            def _pf():
                bp = (b + 2) % 3
                wait_out(bp)
                start_in(nxt, bp)

            if first:
                pl.when(jnp.logical_and(c >= 1, nxt < nch))(_pf)
            else:
                pl.when(nxt < nch)(_pf)

        start_in(0, 0)
        start_in(1, 1)
        start_in(2, 2)

        @pl.loop(0, main, step=3)
        def _chunks(c0):
            for b in range(3):
                step(c0 + b, b, b == 0)

        for c in range(main, nch):
            step(c, c % 3, False)

        wait_out((nch + 2) % 3)
        wait_out((nch + 1) % 3)
        wait_out(nch % 3)

    f32, i32 = jnp.float32, jnp.int32
    return pl.kernel(
        body,
        out_type=jax.ShapeDtypeStruct((ROWS * D_MODEL,), f32),
        mesh=mesh,
        scratch_types=[
            pltpu.VMEM((MAX_LEN * D_MODEL,), f32),
            pltpu.VMEM((C,), i32),
            pltpu.VMEM((C,), i32),
            pltpu.VMEM((C,), i32),
            pltpu.VMEM((CB,), f32),
            pltpu.VMEM((CB,), f32),
            pltpu.VMEM((CB,), f32),
            pltpu.SemaphoreType.DMA,
            pltpu.SemaphoreType.DMA,
            pltpu.SemaphoreType.DMA,
            pltpu.SemaphoreType.DMA,
            pltpu.SemaphoreType.DMA,
            pltpu.SemaphoreType.DMA,
            pltpu.SemaphoreType.DMA,
        ],
    )


def kernel(x, timestamp, te):
    xf = x.reshape(ROWS * D_MODEL)
    tsf = timestamp.astype(jnp.int32).reshape(ROWS)
    tef = te.reshape(MAX_LEN * D_MODEL)
    out = _make_sc_call()(xf, tsf, tef)
    return out.reshape(BATCH, HIST, D_MODEL)


# speedup vs baseline: 3.4107x; 1.2715x over previous
import jax
import jax.numpy as jnp
from jax import lax
from jax.experimental import pallas as pl
from jax.experimental.pallas import tpu as pltpu
from jax.experimental.pallas import tpu_sc as plsc

D_MODEL = 64
MAX_LEN = 446
BATCH = 4096
HIST = 200
ROWS = BATCH * HIST
LANES = 16


def _make_sc_call():
    mesh = plsc.VectorSubcoreMesh(core_axis_name="c", subcore_axis_name="s")
    nc, ns = mesh.num_cores, mesh.num_subcores
    nw = nc * ns
    assert BATCH % nw == 0
    epw = BATCH // nw
    nch = epw
    main = nch - (nch % 3)

    def body(x_hbm, ts_hbm, te_hbm, out_hbm,
             te_v, ts0, ts1, ts2, xb0, xb1, xb2,
             ste, si0, si1, si2, so0, so1, so2):
        wid = lax.axis_index("s") * nc + lax.axis_index("c")
        base = wid * epw

        pltpu.async_copy(te_hbm, te_v, ste).wait()

        bufs = ((xb0, ts0, si0, so0), (xb1, ts1, si1, so1), (xb2, ts2, si2, so2))

        def start_in(c, b):
            xb, tsb, si, _ = bufs[b]
            e = base + c
            pltpu.async_copy(x_hbm.at[e], xb, si)
            pltpu.async_copy(ts_hbm.at[pl.ds(e * HIST, HIST)],
                             tsb.at[pl.ds(0, HIST)], si)

        def wait_in(b):
            xb, tsb, si, _ = bufs[b]
            pltpu.make_async_copy(x_hbm.at[0], xb, si).wait()
            pltpu.make_async_copy(ts_hbm.at[pl.ds(0, HIST)],
                                  tsb.at[pl.ds(0, HIST)], si).wait()

        def start_out(c, b):
            xb, _, _, so = bufs[b]
            e = base + c
            pltpu.async_copy(xb, out_hbm.at[e], so)

        def wait_out(b):
            xb, _, _, so = bufs[b]
            pltpu.make_async_copy(x_hbm.at[0], xb, so).wait()

        def rows4(xb, tbv, j0, jj):
            tev = [te_v[pl.ds(tbv[jj + r] + k, LANES)]
                   for r in range(4)
                   for k in range(0, D_MODEL, LANES)]
            for r in range(4):
                for ki, k in enumerate(range(0, D_MODEL, LANES)):
                    plsc.addupdate(xb.at[j0 + jj + r, pl.ds(k, LANES)],
                                   tev[r * 4 + ki])

        def compute(b):
            xb, tsb, _, _ = bufs[b]

            full = HIST - (HIST % LANES)

            @pl.loop(0, full, step=LANES)
            def _rows(j0):
                tbv = tsb[pl.ds(j0, LANES)] * D_MODEL
                for jj in range(0, LANES, 4):
                    rows4(xb, tbv, j0, jj)

            tbv = tsb[pl.ds(full, LANES)] * D_MODEL
            for jj in range(0, HIST % LANES, 4):
                rows4(xb, tbv, full, jj)

        def step(c, b, first):
            wait_in(b)
            compute(b)
            start_out(c, b)
            nxt = c + 2

            def _pf():
                bp = (b + 2) % 3
                wait_out(bp)
                start_in(nxt, bp)

            if first:
                pl.when(jnp.logical_and(c >= 1, nxt < nch))(_pf)
            else:
                pl.when(nxt < nch)(_pf)

        start_in(0, 0)
        start_in(1, 1)
        start_in(2, 2)

        @pl.loop(0, main, step=3)
        def _chunks(c0):
            for b in range(3):
                step(c0 + b, b, b == 0)

        for c in range(main, nch):
            step(c, c % 3, False)

        wait_out((nch + 2) % 3)
        wait_out((nch + 1) % 3)
        wait_out(nch % 3)

    f32, i32 = jnp.float32, jnp.int32
    return pl.kernel(
        body,
        out_type=jax.ShapeDtypeStruct((BATCH, HIST, D_MODEL), f32),
        mesh=mesh,
        compiler_params=pltpu.CompilerParams(use_tc_tiling_on_sc=True),
        scratch_types=[
            pltpu.VMEM((MAX_LEN * D_MODEL,), f32),
            pltpu.VMEM((HIST + LANES,), i32),
            pltpu.VMEM((HIST + LANES,), i32),
            pltpu.VMEM((HIST + LANES,), i32),
            pltpu.VMEM((HIST, D_MODEL), f32),
            pltpu.VMEM((HIST, D_MODEL), f32),
            pltpu.VMEM((HIST, D_MODEL), f32),
            pltpu.SemaphoreType.DMA,
            pltpu.SemaphoreType.DMA,
            pltpu.SemaphoreType.DMA,
            pltpu.SemaphoreType.DMA,
            pltpu.SemaphoreType.DMA,
            pltpu.SemaphoreType.DMA,
            pltpu.SemaphoreType.DMA,
        ],
    )


def kernel(x, timestamp, te):
    tsf = timestamp.astype(jnp.int32).reshape(ROWS)
    tef = te.reshape(MAX_LEN * D_MODEL)
    return _make_sc_call()(x, tsf, tef)
